# baseline (device time: 11791 ns/iter reference)
import jax
import jax.numpy as jnp
from jax import lax
from jax.experimental import pallas as pl
from jax.experimental.pallas import tpu as pltpu

N_DEV = 4
N_TOK = 256
D_IN = 128
D_OUT = 256
N_EXP = 8
EXP_PER_DEV = N_EXP // N_DEV
CAP = 25


def kernel(x, router_W, route_idx, expert_W):
    def body(x_ref, rw_ref, idx_ref, ew_ref, out_ref,
             send_buf, comm_ref, send_sems, recv_sems):
        my = lax.axis_index("i")
        partners = (my ^ 1, my ^ 3)

        barrier_sem = pltpu.get_barrier_semaphore()
        for nbr in partners:
            pl.semaphore_signal(
                barrier_sem, inc=1,
                device_id=(nbr,), device_id_type=pl.DeviceIdType.MESH,
            )

        idx = idx_ref[:, :]
        e_iota = lax.broadcasted_iota(jnp.int32, (N_TOK, N_EXP), 1)
        onehot = (idx == e_iota).astype(jnp.float32)

        row_i = lax.broadcasted_iota(jnp.int32, (N_TOK, N_TOK), 0)
        col_j = lax.broadcasted_iota(jnp.int32, (N_TOK, N_TOK), 1)
        lower_tri = (col_j <= row_i).astype(jnp.float32)
        cum = jnp.dot(lower_tri, onehot, preferred_element_type=jnp.float32)
        keep = jnp.sum(
            onehot * (cum <= CAP).astype(jnp.float32), axis=1, keepdims=True
        )

        xv = x_ref[:, :]
        e0 = my * EXP_PER_DEV
        g0 = keep * (idx == e0).astype(jnp.float32)
        g1 = keep * (idx == e0 + 1).astype(jnp.float32)
        xg = jnp.concatenate([g0 * xv, g1 * xv], axis=1).astype(jnp.bfloat16)
        w = ew_ref[:, :, :].reshape(EXP_PER_DEV * D_IN, D_OUT).astype(jnp.bfloat16)
        acc = jnp.dot(xg, w, preferred_element_type=jnp.float32)

        for r, partner in enumerate(partners):
            send_buf[:, :] = acc.astype(jnp.bfloat16)
            if r == 0:
                pl.semaphore_wait(barrier_sem, 2)
            rdma = pltpu.make_async_remote_copy(
                src_ref=send_buf,
                dst_ref=comm_ref.at[r],
                send_sem=send_sems.at[r],
                recv_sem=recv_sems.at[r],
                device_id=(partner,),
                device_id_type=pl.DeviceIdType.MESH,
            )
            rdma.start()
            rdma.wait()
            acc = acc + comm_ref[r, :, :].astype(jnp.float32)

        out_ref[:, :] = acc

    return pl.pallas_call(
        body,
        out_shape=jax.ShapeDtypeStruct((N_TOK, D_OUT), jnp.float32),
        in_specs=[pl.BlockSpec(memory_space=pltpu.VMEM)] * 4,
        out_specs=pl.BlockSpec(memory_space=pltpu.VMEM),
        scratch_shapes=[
            pltpu.VMEM((N_TOK, D_OUT), jnp.bfloat16),
            pltpu.VMEM((2, N_TOK, D_OUT), jnp.bfloat16),
            pltpu.SemaphoreType.DMA((2,)),
            pltpu.SemaphoreType.DMA((2,)),
        ],
        compiler_params=pltpu.CompilerParams(collective_id=0),
    )(x, router_W, route_idx, expert_W)


# device time: 10383 ns/iter; 1.1356x vs baseline; 1.1356x over previous
import jax
import jax.numpy as jnp
from jax import lax
from jax.experimental import pallas as pl
from jax.experimental.pallas import tpu as pltpu

N_DEV = 4
N_TOK = 256
D_IN = 128
D_OUT = 256
N_EXP = 8
EXP_PER_DEV = N_EXP // N_DEV
CAP = 25


def kernel(x, router_W, route_idx, expert_W):
    def body(x_ref, rw_ref, idx_ref, ew_ref, out_ref,
             send_buf, comm_ref, send_sems, recv_sems):
        my = lax.axis_index("i")
        partners = (my ^ 1, my ^ 2, my ^ 3)

        barrier_sem = pltpu.get_barrier_semaphore()
        for nbr in partners:
            pl.semaphore_signal(
                barrier_sem, inc=1,
                device_id=(nbr,), device_id_type=pl.DeviceIdType.MESH,
            )

        idx = idx_ref[:, :]
        e_iota = lax.broadcasted_iota(jnp.int32, (N_TOK, N_EXP), 1)
        onehot = (idx == e_iota).astype(jnp.float32)

        row_i = lax.broadcasted_iota(jnp.int32, (N_TOK, N_TOK), 0)
        col_j = lax.broadcasted_iota(jnp.int32, (N_TOK, N_TOK), 1)
        lower_tri = (col_j <= row_i).astype(jnp.float32)
        cum = jnp.dot(lower_tri, onehot, preferred_element_type=jnp.float32)
        keep = jnp.sum(
            onehot * (cum <= CAP).astype(jnp.float32), axis=1, keepdims=True
        )

        xv = x_ref[:, :]
        e0 = my * EXP_PER_DEV
        g0 = keep * (idx == e0).astype(jnp.float32)
        g1 = keep * (idx == e0 + 1).astype(jnp.float32)
        xg = jnp.concatenate([g0 * xv, g1 * xv], axis=1).astype(jnp.bfloat16)
        w = ew_ref[:, :, :].reshape(EXP_PER_DEV * D_IN, D_OUT).astype(jnp.bfloat16)
        acc = jnp.dot(xg, w, preferred_element_type=jnp.float32)

        send_buf[:, :] = acc.astype(jnp.bfloat16)
        pl.semaphore_wait(barrier_sem, len(partners))
        rdmas = []
        for k, partner in enumerate(partners):
            rdma = pltpu.make_async_remote_copy(
                src_ref=send_buf,
                dst_ref=comm_ref.at[k],
                send_sem=send_sems.at[k],
                recv_sem=recv_sems.at[k],
                device_id=(partner,),
                device_id_type=pl.DeviceIdType.MESH,
            )
            rdma.start()
            rdmas.append(rdma)
        for k in (0, 2, 1):
            rdmas[k].wait_recv()
            acc = acc + comm_ref[k, :, :].astype(jnp.float32)
        out_ref[:, :] = acc
        for rdma in rdmas:
            rdma.wait_send()

    return pl.pallas_call(
        body,
        out_shape=jax.ShapeDtypeStruct((N_TOK, D_OUT), jnp.float32),
        in_specs=[pl.BlockSpec(memory_space=pltpu.VMEM)] * 4,
        out_specs=pl.BlockSpec(memory_space=pltpu.VMEM),
        scratch_shapes=[
            pltpu.VMEM((N_TOK, D_OUT), jnp.bfloat16),
            pltpu.VMEM((3, N_TOK, D_OUT), jnp.bfloat16),
            pltpu.SemaphoreType.DMA((3,)),
            pltpu.SemaphoreType.DMA((3,)),
        ],
        compiler_params=pltpu.CompilerParams(collective_id=0),
    )(x, router_W, route_idx, expert_W)


# device time: 8516 ns/iter; 1.3846x vs baseline; 1.2192x over previous
import jax
import jax.numpy as jnp
from jax import lax
from jax.experimental import pallas as pl
from jax.experimental.pallas import tpu as pltpu

N_DEV = 4
N_TOK = 256
D_IN = 128
D_OUT = 256
N_EXP = 8
EXP_PER_DEV = N_EXP // N_DEV
CAP = 25
PAD = 32
BLK = EXP_PER_DEV * PAD
TOT = N_EXP * PAD


def kernel(x, router_W, route_idx, expert_W):
    def body(x_ref, rw_ref, idx_ref, ew_ref, out_ref,
             gath_ref, send_sems, recv_sems):
        my = lax.axis_index("i")
        partners = (my ^ 1, my ^ 2, my ^ 3)

        barrier_sem = pltpu.get_barrier_semaphore()
        for nbr in partners:
            pl.semaphore_signal(
                barrier_sem, inc=1,
                device_id=(nbr,), device_id_type=pl.DeviceIdType.MESH,
            )

        idx = idx_ref[:, :]
        e_iota = lax.broadcasted_iota(jnp.int32, (N_TOK, N_EXP), 1)
        onehot = (idx == e_iota).astype(jnp.float32)
        row_i = lax.broadcasted_iota(jnp.int32, (N_TOK, N_TOK), 0)
        col_j = lax.broadcasted_iota(jnp.int32, (N_TOK, N_TOK), 1)
        lower_tri = (col_j <= row_i).astype(jnp.float32)
        cum = jnp.dot(lower_tri, onehot, preferred_element_type=jnp.float32)

        j_e = lax.broadcasted_iota(jnp.int32, (N_TOK, TOT), 1) // PAD
        j_k = lax.broadcasted_iota(jnp.int32, (N_TOK, TOT), 1) % PAD
        ee = lax.broadcasted_iota(jnp.int32, (N_EXP, TOT), 0)
        ej = lax.broadcasted_iota(jnp.int32, (N_EXP, TOT), 1) // PAD
        e_sel = (ee == ej).astype(jnp.float32)
        cum_j = jnp.dot(cum, e_sel, preferred_element_type=jnp.float32)
        smask = (
            (idx == j_e)
            & (cum_j == (j_k + 1).astype(jnp.float32))
            & (j_k < CAP)
        ).astype(jnp.bfloat16)

        c_k = lax.broadcasted_iota(jnp.int32, (N_TOK, BLK), 1)
        e_loc = my * EXP_PER_DEV + c_k // PAD
        kk = c_k % PAD
        ee_l = lax.broadcasted_iota(jnp.int32, (N_EXP, BLK), 0)
        ej_l = my * EXP_PER_DEV + lax.broadcasted_iota(
            jnp.int32, (N_EXP, BLK), 1) // PAD
        cum_loc = jnp.dot(cum, (ee_l == ej_l).astype(jnp.float32),
                          preferred_element_type=jnp.float32)
        sl = (
            (idx == e_loc)
            & (cum_loc == (kk + 1).astype(jnp.float32))
            & (kk < CAP)
        ).astype(jnp.bfloat16)

        xb = x_ref[:, :].astype(jnp.bfloat16)
        xc = lax.dot_general(
            sl, xb, (((0,), (0,)), ((), ())),
            preferred_element_type=jnp.float32,
        ).astype(jnp.bfloat16)
        y0 = jnp.dot(xc[0:PAD], ew_ref[0].astype(jnp.bfloat16),
                     preferred_element_type=jnp.float32)
        y1 = jnp.dot(xc[PAD:BLK], ew_ref[1].astype(jnp.bfloat16),
                     preferred_element_type=jnp.float32)
        gath_ref[pl.ds(my * BLK, BLK), :] = (
            jnp.concatenate([y0, y1], axis=0).astype(jnp.bfloat16)
        )

        pl.semaphore_wait(barrier_sem, len(partners))
        rdmas = []
        for k, partner in enumerate(partners):
            rdma = pltpu.make_async_remote_copy(
                src_ref=gath_ref.at[pl.ds(my * BLK, BLK), :],
                dst_ref=gath_ref.at[pl.ds(my * BLK, BLK), :],
                send_sem=send_sems.at[k],
                recv_sem=recv_sems.at[k],
                device_id=(partner,),
                device_id_type=pl.DeviceIdType.MESH,
            )
            rdma.start()
            rdmas.append(rdma)
        for k in (0, 2, 1):
            rdmas[k].wait_recv()

        out_ref[:, :] = jnp.dot(smask, gath_ref[:, :],
                                preferred_element_type=jnp.float32)
        for rdma in rdmas:
            rdma.wait_send()

    return pl.pallas_call(
        body,
        out_shape=jax.ShapeDtypeStruct((N_TOK, D_OUT), jnp.float32),
        in_specs=[pl.BlockSpec(memory_space=pltpu.VMEM)] * 4,
        out_specs=pl.BlockSpec(memory_space=pltpu.VMEM),
        scratch_shapes=[
            pltpu.VMEM((TOT, D_OUT), jnp.bfloat16),
            pltpu.SemaphoreType.DMA((3,)),
            pltpu.SemaphoreType.DMA((3,)),
        ],
        compiler_params=pltpu.CompilerParams(collective_id=0),
    )(x, router_W, route_idx, expert_W)
